# baseline (speedup 1.0000x reference)
import functools

import jax
import jax.numpy as jnp
from jax.experimental import pallas as pl
from jax.experimental.pallas import tpu as pltpu


def _embed_kernel(x_ref, w_ref, b_ref, pos_ref, out_ref):
    tb, l, c = x_ref.shape
    tn = w_ref.shape[1]
    x = x_ref[...].reshape(tb * l, c)
    acc = jnp.dot(x, w_ref[...], preferred_element_type=jnp.float32)
    out = acc.reshape(tb, l, tn)
    out = out + pos_ref[...][None, :, :] + b_ref[...][None, None, :]
    out_ref[...] = out


@functools.partial(jax.jit, static_argnames=("tb", "tn", "interpret"))
def _embed(sequence, W_tok, b_tok, pos_table, tb=256, tn=1024, interpret=False):
    B, L, C = sequence.shape
    D = W_tok.shape[1]
    grid = (B // tb, D // tn)
    out = pl.pallas_call(
        _embed_kernel,
        grid=grid,
        in_specs=[
            pl.BlockSpec((tb, L, C), lambda i, j: (i, 0, 0)),
            pl.BlockSpec((C, tn), lambda i, j: (0, j)),
            pl.BlockSpec((tn,), lambda i, j: (j,)),
            pl.BlockSpec((L, tn), lambda i, j: (0, j)),
        ],
        out_specs=pl.BlockSpec((tb, L, tn), lambda i, j: (i, 0, j)),
        out_shape=jax.ShapeDtypeStruct((B, L, D), jnp.float32),
        compiler_params=pltpu.CompilerParams(
            dimension_semantics=("arbitrary", "arbitrary"),
        ),
        interpret=interpret,
    )(sequence, W_tok, b_tok, pos_table)
    return out


def kernel(sequence, W_tok, b_tok, pos_table):
    B, L, C = sequence.shape
    x = _embed(sequence, W_tok, b_tok, pos_table)
    mask = jnp.ones((B, L), dtype=bool)
    return (x, mask)
